# baseline (speedup 1.0000x reference)
import functools

import jax
import jax.numpy as jnp
from jax import lax
from jax.experimental import pallas as pl
from jax.experimental.pallas import tpu as pltpu
from jax.experimental.pallas import tpu_sc as plsc

N_BOT, N_BUF, M, D = 32768, 16384, 16384, 128

NC, NS = 2, 16
NW = NC * NS
CHUNK = 128
ROWS_PER_JOB = M // 2
IDX_ROWS = ROWS_PER_JOB // CHUNK
ROWS_PER_WORKER = ROWS_PER_JOB // NW
CHUNKS_PER_WORKER = ROWS_PER_WORKER // CHUNK
NBUF = 7
LAG = 6


def _sc_route_body(h_bot, c_bot, h_buf, c_buf,
                   bf0, bt0, pf0, pt0, bf1, bt1, pf1, pt1,
                   lh, lc, rh, rc,
                   fidx_v, tidx_v, rows, isem, *sems):
  gsems, ssems = sems[:NBUF], sems[NBUF:]
  wid = lax.axis_index("s") * NC + lax.axis_index("c")
  base = wid * ROWS_PER_WORKER
  idescs = {}
  for fj, (fa, ta) in enumerate([(bf0, bt0), (pf0, pt0), (bf1, bt1),
                                 (pf1, pt1)]):
    ds = []
    for ci in range(CHUNKS_PER_WORKER):
      off = base + ci * CHUNK
      ds.append(pltpu.async_copy(
          fa.at[pl.ds(off, CHUNK)], fidx_v.at[fj, ci], isem))
      ds.append(pltpu.async_copy(
          ta.at[pl.ds(off, CHUNK)], tidx_v.at[fj, ci], isem))
    idescs[fj] = ds

  jobs = [(h_bot, 0, lh), (h_buf, 1, lh), (c_bot, 0, lc), (c_buf, 1, lc),
          (h_bot, 2, rh), (h_buf, 3, rh), (c_bot, 2, rc), (c_buf, 3, rc)]
  tasks = [(t, fj, out, ci) for (t, fj, out) in jobs
           for ci in range(CHUNKS_PER_WORKER)]
  n = len(tasks)
  gd = [None] * NBUF
  sd = [None] * NBUF
  for k in range(n + LAG):
    if k < n:
      table, fj, out, ci = tasks[k]
      slot = k % NBUF
      if idescs.get(fj):
        for d in idescs.pop(fj):
          d.wait()
      if sd[slot] is not None:
        sd[slot].wait()
      gd[slot] = pltpu.async_copy(
          table.at[fidx_v.at[fj, ci]], rows.at[slot], gsems[slot])
    kk = k - LAG
    if 0 <= kk < n:
      table, fj, out, ci = tasks[kk]
      slot = kk % NBUF
      gd[slot].wait()
      sd[slot] = pltpu.async_copy(
          rows.at[slot], out.at[tidx_v.at[fj, ci]], ssems[slot])
  for slot in range(NBUF):
    if sd[slot] is not None:
      sd[slot].wait()


def _sc_route(h_bot, c_bot, h_buf, c_buf, idx8):
  mesh = plsc.VectorSubcoreMesh(core_axis_name="c", subcore_axis_name="s",
                                num_cores=NC, num_subcores=NS)
  out_type = [jax.ShapeDtypeStruct((M, D), jnp.float32) for _ in range(4)]
  scratch = [
      pltpu.VMEM((4, CHUNKS_PER_WORKER, CHUNK), jnp.int32),
      pltpu.VMEM((4, CHUNKS_PER_WORKER, CHUNK), jnp.int32),
      pltpu.VMEM((NBUF, CHUNK, D), jnp.float32),
      pltpu.SemaphoreType.DMA,
  ] + [pltpu.SemaphoreType.DMA] * (2 * NBUF)
  fn = pl.kernel(_sc_route_body, out_type=out_type, mesh=mesh,
                 scratch_types=scratch)
  return fn(h_bot, c_bot, h_buf, c_buf, *idx8)


def _sigmoid(x):
  return 0.5 * jnp.tanh(0.5 * x) + 0.5


def _tc_cell_body(lh, rh, lc, rc, W_iou, b_iou, U_f, b_f, h_out, c_out):
  x = jnp.concatenate([lh[...], rh[...]], axis=1).astype(jnp.bfloat16)
  iou = jnp.dot(x, W_iou[...], preferred_element_type=jnp.float32) + b_iou[...]
  f = _sigmoid(
      jnp.dot(x, U_f[...], preferred_element_type=jnp.float32) + b_f[...])
  i = _sigmoid(iou[:, :D])
  o = _sigmoid(iou[:, D:2 * D])
  u = jnp.tanh(iou[:, 2 * D:])
  c = i * u + f[:, :D] * lc[...] + f[:, D:] * rc[...]
  h_out[...] = o * jnp.tanh(c)
  c_out[...] = c


def _tc_cell(lh, rh, lc, rc, W_iou, b_iou, U_f, b_f, block=4096):
  grid = (M // block,)
  row_spec = pl.BlockSpec((block, D), lambda i: (i, 0))
  full = lambda shape: pl.BlockSpec(shape, lambda i: (0,) * len(shape))
  return pl.pallas_call(
      _tc_cell_body,
      grid=grid,
      in_specs=[row_spec, row_spec, row_spec, row_spec,
                full((2 * D, 3 * D)), full((1, 3 * D)),
                full((2 * D, 2 * D)), full((1, 2 * D))],
      out_specs=[row_spec, row_spec],
      out_shape=[jax.ShapeDtypeStruct((M, D), jnp.float32),
                 jax.ShapeDtypeStruct((M, D), jnp.float32)],
  )(lh, rh, lc, rc, W_iou, b_iou, U_f, b_f)


@jax.jit
def kernel(h_bot, c_bot, h_buf, c_buf,
           bot_froms_0, bot_tos_0, prev_froms_0, prev_tos_0,
           bot_froms_1, bot_tos_1, prev_froms_1, prev_tos_1,
           W_iou, b_iou, U_f, b_f):
  i32 = lambda a: jnp.asarray(a, jnp.int32)
  idx8 = [i32(a) for a in (bot_froms_0, bot_tos_0, prev_froms_0, prev_tos_0,
                           bot_froms_1, bot_tos_1, prev_froms_1, prev_tos_1)]
  lh, lc, rh, rc = _sc_route(h_bot, c_bot, h_buf, c_buf, idx8)
  return _tc_cell(lh, rh, lc, rc, W_iou.astype(jnp.bfloat16),
                  b_iou.reshape(1, -1), U_f.astype(jnp.bfloat16),
                  b_f.reshape(1, -1))


# speedup vs baseline: 1.0075x; 1.0075x over previous
import jax
import jax.numpy as jnp
from jax import lax
from jax.experimental import pallas as pl
from jax.experimental.pallas import tpu as pltpu
from jax.experimental.pallas import tpu_sc as plsc

N_BOT, N_BUF, M, D = 32768, 16384, 16384, 128

NC, NS = 2, 16
NW = NC * NS
CHUNK = 128
ROWS_PER_JOB = M // 2
IDX_ROWS = ROWS_PER_JOB // CHUNK
ROWS_PER_WORKER = ROWS_PER_JOB // NW
CHUNKS_PER_WORKER = ROWS_PER_WORKER // CHUNK
NBUF = 7
LAG = 4


def _sc_route_body(h_bot, c_bot, h_buf, c_buf,
                   bf0, bt0, pf0, pt0, bf1, bt1, pf1, pt1,
                   lh, lc, rh, rc,
                   fidx_v, tidx_v, rows, isem, *sems):
  gsems, ssems = sems[:NBUF], sems[NBUF:]
  wid = lax.axis_index("s") * NC + lax.axis_index("c")
  base = wid * ROWS_PER_WORKER
  idescs = {}
  for fj, (fa, ta) in enumerate([(bf0, bt0), (pf0, pt0), (bf1, bt1),
                                 (pf1, pt1)]):
    ds = []
    for ci in range(CHUNKS_PER_WORKER):
      off = base + ci * CHUNK
      ds.append(pltpu.async_copy(
          fa.at[pl.ds(off, CHUNK)], fidx_v.at[fj, ci], isem))
      ds.append(pltpu.async_copy(
          ta.at[pl.ds(off, CHUNK)], tidx_v.at[fj, ci], isem))
    idescs[fj] = ds

  jobs = [(h_bot, 0, lh), (h_buf, 1, lh), (c_bot, 0, lc), (c_buf, 1, lc),
          (h_bot, 2, rh), (h_buf, 3, rh), (c_bot, 2, rc), (c_buf, 3, rc)]
  tasks = [(t, fj, out, ci) for (t, fj, out) in jobs
           for ci in range(CHUNKS_PER_WORKER)]
  n = len(tasks)
  gd = [None] * NBUF
  sd = [None] * NBUF
  for k in range(n + LAG):
    if k < n:
      table, fj, out, ci = tasks[k]
      slot = k % NBUF
      if idescs.get(fj):
        for d in idescs.pop(fj):
          d.wait()
      if sd[slot] is not None:
        sd[slot].wait()
      gd[slot] = pltpu.async_copy(
          table.at[fidx_v.at[fj, ci]], rows.at[slot], gsems[slot])
    kk = k - LAG
    if 0 <= kk < n:
      table, fj, out, ci = tasks[kk]
      slot = kk % NBUF
      gd[slot].wait()
      sd[slot] = pltpu.async_copy(
          rows.at[slot], out.at[tidx_v.at[fj, ci]], ssems[slot])
  for slot in range(NBUF):
    if sd[slot] is not None:
      sd[slot].wait()


def _sc_route(h_bot, c_bot, h_buf, c_buf, idx8):
  mesh = plsc.VectorSubcoreMesh(core_axis_name="c", subcore_axis_name="s",
                                num_cores=NC, num_subcores=NS)
  out_type = [jax.ShapeDtypeStruct((M, D), jnp.float32) for _ in range(4)]
  scratch = [
      pltpu.VMEM((4, CHUNKS_PER_WORKER, CHUNK), jnp.int32),
      pltpu.VMEM((4, CHUNKS_PER_WORKER, CHUNK), jnp.int32),
      pltpu.VMEM((NBUF, CHUNK, D), jnp.float32),
      pltpu.SemaphoreType.DMA,
  ] + [pltpu.SemaphoreType.DMA] * (2 * NBUF)
  fn = pl.kernel(_sc_route_body, out_type=out_type, mesh=mesh,
                 scratch_types=scratch)
  return fn(h_bot, c_bot, h_buf, c_buf, *idx8)


def _sigmoid(x):
  return 0.5 * jnp.tanh(0.5 * x) + 0.5


def _tc_cell_body(lh, rh, lc, rc, W_iou, b_iou, U_f, b_f, h_out, c_out):
  x = jnp.concatenate([lh[...], rh[...]], axis=1).astype(jnp.bfloat16)
  iou = jnp.dot(x, W_iou[...], preferred_element_type=jnp.float32) + b_iou[...]
  f = _sigmoid(
      jnp.dot(x, U_f[...], preferred_element_type=jnp.float32) + b_f[...])
  i = _sigmoid(iou[:, :D])
  o = _sigmoid(iou[:, D:2 * D])
  u = jnp.tanh(iou[:, 2 * D:])
  c = i * u + f[:, :D] * lc[...] + f[:, D:] * rc[...]
  h_out[...] = o * jnp.tanh(c)
  c_out[...] = c


def _tc_cell(lh, rh, lc, rc, W_iou, b_iou, U_f, b_f, block=4096):
  grid = (M // block,)
  row_spec = pl.BlockSpec((block, D), lambda i: (i, 0))
  full = lambda shape: pl.BlockSpec(shape, lambda i: (0,) * len(shape))
  return pl.pallas_call(
      _tc_cell_body,
      grid=grid,
      in_specs=[row_spec, row_spec, row_spec, row_spec,
                full((2 * D, 3 * D)), full((1, 3 * D)),
                full((2 * D, 2 * D)), full((1, 2 * D))],
      out_specs=[row_spec, row_spec],
      out_shape=[jax.ShapeDtypeStruct((M, D), jnp.float32),
                 jax.ShapeDtypeStruct((M, D), jnp.float32)],
  )(lh, rh, lc, rc, W_iou, b_iou, U_f, b_f)


@jax.jit
def kernel(h_bot, c_bot, h_buf, c_buf,
           bot_froms_0, bot_tos_0, prev_froms_0, prev_tos_0,
           bot_froms_1, bot_tos_1, prev_froms_1, prev_tos_1,
           W_iou, b_iou, U_f, b_f):
  i32 = lambda a: jnp.asarray(a, jnp.int32)
  idx8 = [i32(a) for a in (bot_froms_0, bot_tos_0, prev_froms_0, prev_tos_0,
                           bot_froms_1, bot_tos_1, prev_froms_1, prev_tos_1)]
  lh, lc, rh, rc = _sc_route(h_bot, c_bot, h_buf, c_buf, idx8)
  return _tc_cell(lh, rh, lc, rc, W_iou.astype(jnp.bfloat16),
                  b_iou.reshape(1, -1), U_f.astype(jnp.bfloat16),
                  b_f.reshape(1, -1))
